# baseline (speedup 1.0000x reference)
import jax
import jax.numpy as jnp
from jax import lax
from jax.experimental import pallas as pl
from jax.experimental.pallas import tpu as pltpu
from jax.experimental.pallas import tpu_sc as plsc

N = 100000
E = 6400000

L = 16
SUB = 16
CORES = 2
NW = CORES * SUB
NP = 100352
NR = NP // L
RPT = NP // SUB
QT = RPT // 4
DW = NP // NW
DR = DW // L

W = 1024
KB = 2
G = 98
EPW = W * KB * G
EPAD = EPW * NW

_mesh = plsc.VectorSubcoreMesh(core_axis_name="c", subcore_axis_name="s")
_params = pltpu.CompilerParams(needs_layout_passes=False,
                               use_tc_tiling_on_sc=False)
_f32 = jnp.float32
_i32 = jnp.int32


def _rsqrt16(d):
    i = plsc.bitcast(d, _i32)
    i = jnp.int32(0x5F3759DF) - (i >> 1)
    y = plsc.bitcast(i, _f32)
    for _ in range(3):
        y = y * (1.5 - 0.5 * d * y * y)
    return y


def _ids():
    cid = lax.axis_index("c")
    sid = lax.axis_index("s")
    return cid, sid, cid * SUB + sid


def _edge_pass(comb_h, table_h, shared_acc, idx0, idx1, msg0, msg1,
               sem0, sem1, wid):
    grp_base = wid * G

    def load_idx(g, idx):
        pltpu.sync_copy(comb_h.at[pl.ds((grp_base + g) * 2 * KB, 2 * KB)],
                        idx)

    def fire_gathers(idx, msg, sem):
        return [pltpu.async_copy(table_h.at[idx.at[j]], msg.at[j], sem)
                for j in range(KB)]

    def wait_gathers(idx, msg, sem):
        for j in range(KB):
            pltpu.make_async_copy(table_h.at[idx.at[j]], msg.at[j],
                                  sem).wait()

    def fire_scatters(idx, msg, sem):
        return [pltpu.async_copy(msg.at[j], shared_acc.at[idx.at[KB + j]],
                                 sem, add=True)
                for j in range(KB)]

    def wait(descs):
        for d in descs:
            d.wait()

    load_idx(0, idx0)
    fire_gathers(idx0, msg0, sem0)

    @pl.loop(0, G // 2)
    def _(t):
        ga = 2 * t
        load_idx(ga + 1, idx1)
        fire_gathers(idx1, msg1, sem1)
        wait_gathers(idx0, msg0, sem0)
        wait(fire_scatters(idx0, msg0, sem0))

        @pl.when(t < G // 2 - 1)
        def _():
            load_idx(ga + 2, idx0)
            fire_gathers(idx0, msg0, sem0)

        wait_gathers(idx1, msg1, sem1)
        wait(fire_scatters(idx1, msg1, sem1))


def _acc_writeback(shared_acc, stage, out_ref, r0, out_base):
    for q in range(4):
        pltpu.sync_copy(shared_acc.at[pl.ds(r0 + q * QT, QT)], stage)
        pltpu.sync_copy(stage, out_ref.at[pl.ds(out_base + q * QT, QT)])


def _zero_fill(stage, nrows, ncols):
    iota = lax.iota(_i32, L)
    zero = jnp.zeros((L,), _f32)
    nit = nrows * ncols // L

    @pl.loop(0, nit)
    def _(i):
        f = iota + i * L
        plsc.store_scatter(stage, [f // ncols, f % ncols], zero)


def _deg_body(dst_h, deg_out, shared_deg, ones8, idx_d, stage, cmp_, sem):
    cid, sid, wid = _ids()
    r0 = sid * RPT
    iota = lax.iota(_i32, L)

    _zero_fill(ones8, W, 8)
    one = jnp.full((L,), 1.0, _f32)
    zcol = jnp.zeros((L,), _i32)

    @pl.loop(0, W // L)
    def _(t):
        plsc.store_scatter(ones8, [iota + t * L, zcol], one)

    _zero_fill(stage, QT, 8)
    vinit = one * jnp.where(cid == 0, 1.0, 0.0).astype(_f32)

    @pl.loop(0, QT // L)
    def _(i):
        plsc.store_scatter(stage, [iota + i * L, zcol], vinit)

    for q in range(4):
        pltpu.sync_copy(stage, shared_deg.at[pl.ds(r0 + q * QT, QT)])
    plsc.subcore_barrier()

    row_base = wid * G * KB

    @pl.loop(0, G)
    def _(g):
        rb = row_base + g * KB
        pltpu.sync_copy(dst_h.at[pl.ds(rb, KB)], idx_d)
        puts = [pltpu.async_copy(ones8, shared_deg.at[idx_d.at[j]], sem,
                                 add=True)
                for j in range(KB)]
        for d in puts:
            d.wait()

    plsc.subcore_barrier()

    for q in range(4):
        pltpu.sync_copy(shared_deg.at[pl.ds(r0 + q * QT, QT)], stage)

        @pl.loop(0, QT // L)
        def _(i):
            v = plsc.load_gather(stage, [iota + i * L, zcol])
            cmp_[i] = v

        row_out = cid * NR + sid * (RPT // L) + q * (QT // L)
        pltpu.sync_copy(cmp_, deg_out.at[pl.ds(row_out, QT // L)])


def _dense1_body(x0_h, x1_h, deg_h, wvec_h, h1s_out, dis_out,
                 xb0, xb1, db0, db1, disb, hst, wbuf):
    cid, sid, wid = _ids()
    rw = wid * DR
    sl = pl.ds(rw, DR)
    pltpu.sync_copy(wvec_h, wbuf)
    pltpu.sync_copy(x0_h.at[sl], xb0)
    pltpu.sync_copy(x1_h.at[sl], xb1)
    pltpu.sync_copy(deg_h.at[sl], db0)
    pltpu.sync_copy(deg_h.at[pl.ds(NR + rw, DR)], db1)

    iota = lax.iota(_i32, L)
    wv0 = wbuf[0]
    _zero_fill(hst, DW, 8)

    @pl.loop(0, DR)
    def _(i):
        d = db0[i] + db1[i]
        y = _rsqrt16(d)
        disb[i] = y
        xv0 = xb0[i]
        xv1 = xb1[i]
        rows = iota + i * L
        for j in range(4):
            hj = (xv0 * wv0[j] + xv1 * wv0[4 + j]) * y
            plsc.store_scatter(hst, [rows, jnp.full((L,), j, _i32)], hj)

    pltpu.sync_copy(disb, dis_out.at[sl])
    pltpu.sync_copy(hst, h1s_out.at[pl.ds(wid * DW, DW)])


def _edgek_body(table_h, comb_h, acc_out,
                shared_acc, idx0, idx1, msg0, msg1, stage, sem0, sem1):
    cid, sid, wid = _ids()
    r0 = sid * RPT
    _zero_fill(stage, QT, 8)
    for q in range(4):
        pltpu.sync_copy(stage, shared_acc.at[pl.ds(r0 + q * QT, QT)])
    plsc.subcore_barrier()
    _edge_pass(comb_h, table_h, shared_acc, idx0, idx1, msg0, msg1,
               sem0, sem1, wid)
    plsc.subcore_barrier()
    _acc_writeback(shared_acc, stage, acc_out, r0, cid * NP + r0)


def _dense2_body(x0_h, x1_h, dis_h, acc1_h, wvec_h, h2s_out,
                 xb0, xb1, disb, a0b, a1b, h2st, wbuf):
    cid, sid, wid = _ids()
    rw = wid * DR
    sl = pl.ds(rw, DR)
    nb = wid * DW
    pltpu.sync_copy(wvec_h, wbuf)
    pltpu.sync_copy(x0_h.at[sl], xb0)
    pltpu.sync_copy(x1_h.at[sl], xb1)
    pltpu.sync_copy(dis_h.at[sl], disb)
    pltpu.sync_copy(acc1_h.at[pl.ds(nb, DW)], a0b)
    pltpu.sync_copy(acc1_h.at[pl.ds(NP + nb, DW)], a1b)

    iota = lax.iota(_i32, L)
    zero = jnp.zeros((L,), _f32)
    wv0 = wbuf[0]
    wv1 = wbuf[1]
    _zero_fill(h2st, DW, 8)

    @pl.loop(0, DR)
    def _(i):
        y = disb[i]
        xv0 = xb0[i]
        xv1 = xb1[i]
        rows = iota + i * L
        h2 = [zero, zero]
        for j in range(4):
            colj = jnp.full((L,), j, _i32)
            a = (plsc.load_gather(a0b, [rows, colj])
                 + plsc.load_gather(a1b, [rows, colj]))
            hj = (xv0 * wv0[j] + xv1 * wv0[4 + j]) * y
            o = jnp.maximum((a + hj) * y + wv1[j], 0.0)
            for k in range(2):
                h2[k] = h2[k] + o * wv0[8 + j * 2 + k]
        for k in range(2):
            plsc.store_scatter(h2st, [rows, jnp.full((L,), k, _i32)],
                               h2[k] * y)

    pltpu.sync_copy(h2st, h2s_out.at[pl.ds(nb, DW)])


def _final_body(acc2_h, h2s_h, dis_h, wvec_h, out_r,
                a0b, a1b, h2b, disb, stage, wbuf):
    cid, sid, wid = _ids()
    rw = wid * DR
    nb = wid * DW
    pltpu.sync_copy(wvec_h, wbuf)
    pltpu.sync_copy(acc2_h.at[pl.ds(nb, DW)], a0b)
    pltpu.sync_copy(acc2_h.at[pl.ds(NP + nb, DW)], a1b)
    pltpu.sync_copy(h2s_h.at[pl.ds(nb, DW)], h2b)
    pltpu.sync_copy(dis_h.at[pl.ds(rw, DR)], disb)

    wv1 = wbuf[1]
    iota = lax.iota(_i32, L)

    @pl.loop(0, DR)
    def _(i):
        y = disb[i]
        rows = iota + i * L
        for k in range(2):
            colk = jnp.full((L,), k, _i32)
            v = (plsc.load_gather(a0b, [rows, colk])
                 + plsc.load_gather(a1b, [rows, colk])
                 + plsc.load_gather(h2b, [rows, colk])) * y + wv1[4 + k]
            p = rows * 2 + k
            plsc.store_scatter(stage, [p >> 4, p & 15], v)

    pltpu.sync_copy(stage, out_r.at[pl.ds(wid * (2 * DW // L), 2 * DW // L)])


_deg_kernel = pl.kernel(
    _deg_body,
    compiler_params=_params,
    out_type=jax.ShapeDtypeStruct((CORES * NR, L), _f32),
    mesh=_mesh,
    scratch_types=[
        pltpu.VMEM_SHARED((NP, 8), _f32),
        pltpu.VMEM((W, 8), _f32),
        pltpu.VMEM((KB, W), _i32),
        pltpu.VMEM((QT, 8), _f32),
        pltpu.VMEM((QT // L, L), _f32),
        pltpu.SemaphoreType.DMA,
    ],
)

_dense1_kernel = pl.kernel(
    _dense1_body,
    compiler_params=_params,
    out_type=(jax.ShapeDtypeStruct((NP, 8), _f32),
              jax.ShapeDtypeStruct((NR, L), _f32)),
    mesh=_mesh,
    scratch_types=[
        pltpu.VMEM((DR, L), _f32),
        pltpu.VMEM((DR, L), _f32),
        pltpu.VMEM((DR, L), _f32),
        pltpu.VMEM((DR, L), _f32),
        pltpu.VMEM((DR, L), _f32),
        pltpu.VMEM((DW, 8), _f32),
        pltpu.VMEM((2, L), _f32),
    ],
)

_edgek_kernel = pl.kernel(
    _edgek_body,
    compiler_params=_params,
    out_type=jax.ShapeDtypeStruct((CORES * NP, 8), _f32),
    mesh=_mesh,
    scratch_types=[
        pltpu.VMEM_SHARED((NP, 8), _f32),
        pltpu.VMEM((2 * KB, W), _i32),
        pltpu.VMEM((2 * KB, W), _i32),
        pltpu.VMEM((KB, W, 8), _f32),
        pltpu.VMEM((KB, W, 8), _f32),
        pltpu.VMEM((QT, 8), _f32),
        pltpu.SemaphoreType.DMA,
        pltpu.SemaphoreType.DMA,
    ],
)

_dense2_kernel = pl.kernel(
    _dense2_body,
    compiler_params=_params,
    out_type=jax.ShapeDtypeStruct((NP, 8), _f32),
    mesh=_mesh,
    scratch_types=[
        pltpu.VMEM((DR, L), _f32),
        pltpu.VMEM((DR, L), _f32),
        pltpu.VMEM((DR, L), _f32),
        pltpu.VMEM((DW, 8), _f32),
        pltpu.VMEM((DW, 8), _f32),
        pltpu.VMEM((DW, 8), _f32),
        pltpu.VMEM((2, L), _f32),
    ],
)

_final_kernel = pl.kernel(
    _final_body,
    compiler_params=_params,
    out_type=jax.ShapeDtypeStruct((2 * NP // L, L), _f32),
    mesh=_mesh,
    scratch_types=[
        pltpu.VMEM((DW, 8), _f32),
        pltpu.VMEM((DW, 8), _f32),
        pltpu.VMEM((DW, 8), _f32),
        pltpu.VMEM((DR, L), _f32),
        pltpu.VMEM((2 * DW // L, L), _f32),
        pltpu.VMEM((2, L), _f32),
    ],
)


@jax.jit
def kernel(x, edge_index, W1, b1, W2, b2):
    ei = edge_index.astype(_i32)
    pad = jnp.full((EPAD - E,), N, _i32)
    srcw = jnp.concatenate([ei[0], pad]).reshape(NW * G, KB, W)
    dstw = jnp.concatenate([ei[1], pad]).reshape(NW * G, KB, W)
    comb = jnp.concatenate([srcw, dstw], axis=1).reshape(NW * G * 2 * KB, W)
    dst512 = dstw.reshape(NW * G * KB, W)

    xp = jnp.pad(x.astype(_f32), ((0, NP - N), (0, 0)))
    x0 = xp[:, 0].reshape(NR, L)
    x1 = xp[:, 1].reshape(NR, L)
    wvec = jnp.concatenate([
        W1.reshape(-1), W2.reshape(-1), b1.reshape(-1), b2.reshape(-1),
        jnp.zeros((10,), _f32)]).astype(_f32).reshape(2, L)

    deg2 = _deg_kernel(dst512)
    h1s, dis = _dense1_kernel(x0, x1, deg2, wvec)
    acc1 = _edgek_kernel(h1s, comb)
    h2s = _dense2_kernel(x0, x1, dis, acc1, wvec)
    acc2 = _edgek_kernel(h2s, comb)
    out_r = _final_kernel(acc2, h2s, dis, wvec)
    return out_r.reshape(NP, 2)[:N]


# speedup vs baseline: 1.0013x; 1.0013x over previous
import jax
import jax.numpy as jnp
from jax import lax
from jax.experimental import pallas as pl
from jax.experimental.pallas import tpu as pltpu
from jax.experimental.pallas import tpu_sc as plsc

N = 100000
E = 6400000

L = 16
SUB = 16
CORES = 2
NW = CORES * SUB
NP = 100352
NR = NP // L
RPT = NP // SUB
QT = RPT // 4
DW = NP // NW
DR = DW // L

W = 512
KB = 4
G = 98
EPW = W * KB * G
EPAD = EPW * NW

_mesh = plsc.VectorSubcoreMesh(core_axis_name="c", subcore_axis_name="s")
_params = pltpu.CompilerParams(needs_layout_passes=False,
                               use_tc_tiling_on_sc=False)
_f32 = jnp.float32
_i32 = jnp.int32


def _rsqrt16(d):
    i = plsc.bitcast(d, _i32)
    i = jnp.int32(0x5F3759DF) - (i >> 1)
    y = plsc.bitcast(i, _f32)
    for _ in range(3):
        y = y * (1.5 - 0.5 * d * y * y)
    return y


def _ids():
    cid = lax.axis_index("c")
    sid = lax.axis_index("s")
    return cid, sid, cid * SUB + sid


def _edge_pass(comb_h, table_h, shared_acc, idx0, idx1, msg0, msg1,
               sem0, sem1, wid):
    grp_base = wid * G

    def load_idx(g, idx):
        pltpu.sync_copy(comb_h.at[pl.ds((grp_base + g) * 2 * KB, 2 * KB)],
                        idx)

    def fire_gathers(idx, msg, sem):
        return [pltpu.async_copy(table_h.at[idx.at[j]], msg.at[j], sem)
                for j in range(KB)]

    def wait_gathers(idx, msg, sem):
        for j in range(KB):
            pltpu.make_async_copy(table_h.at[idx.at[j]], msg.at[j],
                                  sem).wait()

    def fire_scatters(idx, msg, sem):
        return [pltpu.async_copy(msg.at[j], shared_acc.at[idx.at[KB + j]],
                                 sem, add=True)
                for j in range(KB)]

    def wait(descs):
        for d in descs:
            d.wait()

    load_idx(0, idx0)
    fire_gathers(idx0, msg0, sem0)

    @pl.loop(0, G // 2)
    def _(t):
        ga = 2 * t
        load_idx(ga + 1, idx1)
        fire_gathers(idx1, msg1, sem1)
        wait_gathers(idx0, msg0, sem0)
        wait(fire_scatters(idx0, msg0, sem0))

        @pl.when(t < G // 2 - 1)
        def _():
            load_idx(ga + 2, idx0)
            fire_gathers(idx0, msg0, sem0)

        wait_gathers(idx1, msg1, sem1)
        wait(fire_scatters(idx1, msg1, sem1))


def _acc_writeback(shared_acc, stage, out_ref, r0, out_base):
    for q in range(4):
        pltpu.sync_copy(shared_acc.at[pl.ds(r0 + q * QT, QT)], stage)
        pltpu.sync_copy(stage, out_ref.at[pl.ds(out_base + q * QT, QT)])


def _zero_fill(stage, nrows, ncols):
    iota = lax.iota(_i32, L)
    zero = jnp.zeros((L,), _f32)
    nit = nrows * ncols // L

    @pl.loop(0, nit)
    def _(i):
        f = iota + i * L
        plsc.store_scatter(stage, [f // ncols, f % ncols], zero)


def _deg_body(dst_h, deg_out, shared_deg, ones8, idx_d, stage, cmp_, sem):
    cid, sid, wid = _ids()
    r0 = sid * RPT
    iota = lax.iota(_i32, L)

    _zero_fill(ones8, W, 8)
    one = jnp.full((L,), 1.0, _f32)
    zcol = jnp.zeros((L,), _i32)

    @pl.loop(0, W // L)
    def _(t):
        plsc.store_scatter(ones8, [iota + t * L, zcol], one)

    _zero_fill(stage, QT, 8)
    vinit = one * jnp.where(cid == 0, 1.0, 0.0).astype(_f32)

    @pl.loop(0, QT // L)
    def _(i):
        plsc.store_scatter(stage, [iota + i * L, zcol], vinit)

    for q in range(4):
        pltpu.sync_copy(stage, shared_deg.at[pl.ds(r0 + q * QT, QT)])
    plsc.subcore_barrier()

    row_base = wid * G * KB

    @pl.loop(0, G)
    def _(g):
        rb = row_base + g * KB
        pltpu.sync_copy(dst_h.at[pl.ds(rb, KB)], idx_d)
        puts = [pltpu.async_copy(ones8, shared_deg.at[idx_d.at[j]], sem,
                                 add=True)
                for j in range(KB)]
        for d in puts:
            d.wait()

    plsc.subcore_barrier()

    for q in range(4):
        pltpu.sync_copy(shared_deg.at[pl.ds(r0 + q * QT, QT)], stage)

        @pl.loop(0, QT // L)
        def _(i):
            v = plsc.load_gather(stage, [iota + i * L, zcol])
            cmp_[i] = v

        row_out = cid * NR + sid * (RPT // L) + q * (QT // L)
        pltpu.sync_copy(cmp_, deg_out.at[pl.ds(row_out, QT // L)])


def _dense1_body(x0_h, x1_h, deg_h, wvec_h, h1s_out, dis_out,
                 xb0, xb1, db0, db1, disb, hst, wbuf):
    cid, sid, wid = _ids()
    rw = wid * DR
    sl = pl.ds(rw, DR)
    pltpu.sync_copy(wvec_h, wbuf)
    pltpu.sync_copy(x0_h.at[sl], xb0)
    pltpu.sync_copy(x1_h.at[sl], xb1)
    pltpu.sync_copy(deg_h.at[sl], db0)
    pltpu.sync_copy(deg_h.at[pl.ds(NR + rw, DR)], db1)

    iota = lax.iota(_i32, L)
    wv0 = wbuf[0]
    _zero_fill(hst, DW, 8)

    @pl.loop(0, DR)
    def _(i):
        d = db0[i] + db1[i]
        y = _rsqrt16(d)
        disb[i] = y
        xv0 = xb0[i]
        xv1 = xb1[i]
        rows = iota + i * L
        for j in range(4):
            hj = (xv0 * wv0[j] + xv1 * wv0[4 + j]) * y
            plsc.store_scatter(hst, [rows, jnp.full((L,), j, _i32)], hj)

    pltpu.sync_copy(disb, dis_out.at[sl])
    pltpu.sync_copy(hst, h1s_out.at[pl.ds(wid * DW, DW)])


def _edgek_body(table_h, comb_h, acc_out,
                shared_acc, idx0, idx1, msg0, msg1, stage, sem0, sem1):
    cid, sid, wid = _ids()
    r0 = sid * RPT
    _zero_fill(stage, QT, 8)
    for q in range(4):
        pltpu.sync_copy(stage, shared_acc.at[pl.ds(r0 + q * QT, QT)])
    plsc.subcore_barrier()
    _edge_pass(comb_h, table_h, shared_acc, idx0, idx1, msg0, msg1,
               sem0, sem1, wid)
    plsc.subcore_barrier()
    _acc_writeback(shared_acc, stage, acc_out, r0, cid * NP + r0)


def _dense2_body(x0_h, x1_h, dis_h, acc1_h, wvec_h, h2s_out,
                 xb0, xb1, disb, a0b, a1b, h2st, wbuf):
    cid, sid, wid = _ids()
    rw = wid * DR
    sl = pl.ds(rw, DR)
    nb = wid * DW
    pltpu.sync_copy(wvec_h, wbuf)
    pltpu.sync_copy(x0_h.at[sl], xb0)
    pltpu.sync_copy(x1_h.at[sl], xb1)
    pltpu.sync_copy(dis_h.at[sl], disb)
    pltpu.sync_copy(acc1_h.at[pl.ds(nb, DW)], a0b)
    pltpu.sync_copy(acc1_h.at[pl.ds(NP + nb, DW)], a1b)

    iota = lax.iota(_i32, L)
    zero = jnp.zeros((L,), _f32)
    wv0 = wbuf[0]
    wv1 = wbuf[1]
    _zero_fill(h2st, DW, 8)

    @pl.loop(0, DR)
    def _(i):
        y = disb[i]
        xv0 = xb0[i]
        xv1 = xb1[i]
        rows = iota + i * L
        h2 = [zero, zero]
        for j in range(4):
            colj = jnp.full((L,), j, _i32)
            a = (plsc.load_gather(a0b, [rows, colj])
                 + plsc.load_gather(a1b, [rows, colj]))
            hj = (xv0 * wv0[j] + xv1 * wv0[4 + j]) * y
            o = jnp.maximum((a + hj) * y + wv1[j], 0.0)
            for k in range(2):
                h2[k] = h2[k] + o * wv0[8 + j * 2 + k]
        for k in range(2):
            plsc.store_scatter(h2st, [rows, jnp.full((L,), k, _i32)],
                               h2[k] * y)

    pltpu.sync_copy(h2st, h2s_out.at[pl.ds(nb, DW)])


def _final_body(acc2_h, h2s_h, dis_h, wvec_h, out_r,
                a0b, a1b, h2b, disb, stage, wbuf):
    cid, sid, wid = _ids()
    rw = wid * DR
    nb = wid * DW
    pltpu.sync_copy(wvec_h, wbuf)
    pltpu.sync_copy(acc2_h.at[pl.ds(nb, DW)], a0b)
    pltpu.sync_copy(acc2_h.at[pl.ds(NP + nb, DW)], a1b)
    pltpu.sync_copy(h2s_h.at[pl.ds(nb, DW)], h2b)
    pltpu.sync_copy(dis_h.at[pl.ds(rw, DR)], disb)

    wv1 = wbuf[1]
    iota = lax.iota(_i32, L)

    @pl.loop(0, DR)
    def _(i):
        y = disb[i]
        rows = iota + i * L
        for k in range(2):
            colk = jnp.full((L,), k, _i32)
            v = (plsc.load_gather(a0b, [rows, colk])
                 + plsc.load_gather(a1b, [rows, colk])
                 + plsc.load_gather(h2b, [rows, colk])) * y + wv1[4 + k]
            p = rows * 2 + k
            plsc.store_scatter(stage, [p >> 4, p & 15], v)

    pltpu.sync_copy(stage, out_r.at[pl.ds(wid * (2 * DW // L), 2 * DW // L)])


_deg_kernel = pl.kernel(
    _deg_body,
    compiler_params=_params,
    out_type=jax.ShapeDtypeStruct((CORES * NR, L), _f32),
    mesh=_mesh,
    scratch_types=[
        pltpu.VMEM_SHARED((NP, 8), _f32),
        pltpu.VMEM((W, 8), _f32),
        pltpu.VMEM((KB, W), _i32),
        pltpu.VMEM((QT, 8), _f32),
        pltpu.VMEM((QT // L, L), _f32),
        pltpu.SemaphoreType.DMA,
    ],
)

_dense1_kernel = pl.kernel(
    _dense1_body,
    compiler_params=_params,
    out_type=(jax.ShapeDtypeStruct((NP, 8), _f32),
              jax.ShapeDtypeStruct((NR, L), _f32)),
    mesh=_mesh,
    scratch_types=[
        pltpu.VMEM((DR, L), _f32),
        pltpu.VMEM((DR, L), _f32),
        pltpu.VMEM((DR, L), _f32),
        pltpu.VMEM((DR, L), _f32),
        pltpu.VMEM((DR, L), _f32),
        pltpu.VMEM((DW, 8), _f32),
        pltpu.VMEM((2, L), _f32),
    ],
)

_edgek_kernel = pl.kernel(
    _edgek_body,
    compiler_params=_params,
    out_type=jax.ShapeDtypeStruct((CORES * NP, 8), _f32),
    mesh=_mesh,
    scratch_types=[
        pltpu.VMEM_SHARED((NP, 8), _f32),
        pltpu.VMEM((2 * KB, W), _i32),
        pltpu.VMEM((2 * KB, W), _i32),
        pltpu.VMEM((KB, W, 8), _f32),
        pltpu.VMEM((KB, W, 8), _f32),
        pltpu.VMEM((QT, 8), _f32),
        pltpu.SemaphoreType.DMA,
        pltpu.SemaphoreType.DMA,
    ],
)

_dense2_kernel = pl.kernel(
    _dense2_body,
    compiler_params=_params,
    out_type=jax.ShapeDtypeStruct((NP, 8), _f32),
    mesh=_mesh,
    scratch_types=[
        pltpu.VMEM((DR, L), _f32),
        pltpu.VMEM((DR, L), _f32),
        pltpu.VMEM((DR, L), _f32),
        pltpu.VMEM((DW, 8), _f32),
        pltpu.VMEM((DW, 8), _f32),
        pltpu.VMEM((DW, 8), _f32),
        pltpu.VMEM((2, L), _f32),
    ],
)

_final_kernel = pl.kernel(
    _final_body,
    compiler_params=_params,
    out_type=jax.ShapeDtypeStruct((2 * NP // L, L), _f32),
    mesh=_mesh,
    scratch_types=[
        pltpu.VMEM((DW, 8), _f32),
        pltpu.VMEM((DW, 8), _f32),
        pltpu.VMEM((DW, 8), _f32),
        pltpu.VMEM((DR, L), _f32),
        pltpu.VMEM((2 * DW // L, L), _f32),
        pltpu.VMEM((2, L), _f32),
    ],
)


@jax.jit
def kernel(x, edge_index, W1, b1, W2, b2):
    ei = edge_index.astype(_i32)
    pad = jnp.full((EPAD - E,), N, _i32)
    srcw = jnp.concatenate([ei[0], pad]).reshape(NW * G, KB, W)
    dstw = jnp.concatenate([ei[1], pad]).reshape(NW * G, KB, W)
    comb = jnp.concatenate([srcw, dstw], axis=1).reshape(NW * G * 2 * KB, W)
    dst512 = dstw.reshape(NW * G * KB, W)

    xp = jnp.pad(x.astype(_f32), ((0, NP - N), (0, 0)))
    x0 = xp[:, 0].reshape(NR, L)
    x1 = xp[:, 1].reshape(NR, L)
    wvec = jnp.concatenate([
        W1.reshape(-1), W2.reshape(-1), b1.reshape(-1), b2.reshape(-1),
        jnp.zeros((10,), _f32)]).astype(_f32).reshape(2, L)

    deg2 = _deg_kernel(dst512)
    h1s, dis = _dense1_kernel(x0, x1, deg2, wvec)
    acc1 = _edgek_kernel(h1s, comb)
    h2s = _dense2_kernel(x0, x1, dis, acc1, wvec)
    acc2 = _edgek_kernel(h2s, comb)
    out_r = _final_kernel(acc2, h2s, dis, wvec)
    return out_r.reshape(NP, 2)[:N]


# speedup vs baseline: 1.0616x; 1.0603x over previous
import jax
import jax.numpy as jnp
from jax import lax
from jax.experimental import pallas as pl
from jax.experimental.pallas import tpu as pltpu
from jax.experimental.pallas import tpu_sc as plsc

N = 100000
E = 6400000

L = 16
SUB = 16
CORES = 2
NW = CORES * SUB
NP = 100352
NR = NP // L
RPT = NP // SUB
QT = RPT // 4
DW = NP // NW
DR = DW // L

W = 512
KB = 4
G = 98
EPW = W * KB * G
EPAD = EPW * NW

_mesh = plsc.VectorSubcoreMesh(core_axis_name="c", subcore_axis_name="s")
_params = pltpu.CompilerParams(needs_layout_passes=False,
                               use_tc_tiling_on_sc=False)
_f32 = jnp.float32
_i32 = jnp.int32


def _rsqrt16(d):
    i = plsc.bitcast(d, _i32)
    i = jnp.int32(0x5F3759DF) - (i >> 1)
    y = plsc.bitcast(i, _f32)
    for _ in range(3):
        y = y * (1.5 - 0.5 * d * y * y)
    return y


def _ids():
    cid = lax.axis_index("c")
    sid = lax.axis_index("s")
    return cid, sid, cid * SUB + sid


def _edge_pass(comb_h, table_h, shared_acc, idx0, idx1, msg0, msg1,
               sem0, sem1, wid):
    grp_base = wid * G

    def load_idx(g, idx):
        pltpu.sync_copy(comb_h.at[pl.ds((grp_base + g) * 2 * KB, 2 * KB)],
                        idx)

    def fire_gathers(idx, msg, sem):
        return [pltpu.async_copy(table_h.at[idx.at[j]], msg.at[j], sem)
                for j in range(KB)]

    def wait_gathers(idx, msg, sem):
        for j in range(KB):
            pltpu.make_async_copy(table_h.at[idx.at[j]], msg.at[j],
                                  sem).wait()

    def fire_scatters(idx, msg, sem):
        return [pltpu.async_copy(msg.at[j], shared_acc.at[idx.at[KB + j]],
                                 sem, add=True)
                for j in range(KB)]

    def wait(descs):
        for d in descs:
            d.wait()

    load_idx(0, idx0)
    fire_gathers(idx0, msg0, sem0)

    @pl.loop(0, G // 2)
    def _(t):
        ga = 2 * t
        load_idx(ga + 1, idx1)
        fire_gathers(idx1, msg1, sem1)
        wait_gathers(idx0, msg0, sem0)
        wait(fire_scatters(idx0, msg0, sem0))

        @pl.when(t < G // 2 - 1)
        def _():
            load_idx(ga + 2, idx0)
            fire_gathers(idx0, msg0, sem0)

        wait_gathers(idx1, msg1, sem1)
        wait(fire_scatters(idx1, msg1, sem1))


def _acc_writeback(shared_acc, stage, out_ref, r0, out_base):
    for q in range(4):
        pltpu.sync_copy(shared_acc.at[pl.ds(r0 + q * QT, QT)], stage)
        pltpu.sync_copy(stage, out_ref.at[pl.ds(out_base + q * QT, QT)])


def _zero_fill(stage, nrows, ncols):
    iota = lax.iota(_i32, L)
    zero = jnp.zeros((L,), _f32)
    nit = nrows * ncols // L

    @pl.loop(0, nit)
    def _(i):
        f = iota + i * L
        plsc.store_scatter(stage, [f // ncols, f % ncols], zero)


def _deg_body(dst_h, deg_out, shared_deg, ones8, idx_d, stage, cmp_, sem):
    cid, sid, wid = _ids()
    r0 = sid * RPT
    iota = lax.iota(_i32, L)

    _zero_fill(ones8, W, 8)
    one = jnp.full((L,), 1.0, _f32)
    zcol = jnp.zeros((L,), _i32)

    @pl.loop(0, W // L)
    def _(t):
        plsc.store_scatter(ones8, [iota + t * L, zcol], one)

    _zero_fill(stage, QT, 8)
    vinit = one * jnp.where(cid == 0, 1.0, 0.0).astype(_f32)

    @pl.loop(0, QT // L)
    def _(i):
        plsc.store_scatter(stage, [iota + i * L, zcol], vinit)

    for q in range(4):
        pltpu.sync_copy(stage, shared_deg.at[pl.ds(r0 + q * QT, QT)])
    plsc.subcore_barrier()

    row_base = wid * G * KB

    @pl.loop(0, G)
    def _(g):
        rb = row_base + g * KB
        pltpu.sync_copy(dst_h.at[pl.ds(rb, KB)], idx_d)
        puts = [pltpu.async_copy(ones8, shared_deg.at[idx_d.at[j]], sem,
                                 add=True)
                for j in range(KB)]
        for d in puts:
            d.wait()

    plsc.subcore_barrier()

    for q in range(4):
        pltpu.sync_copy(shared_deg.at[pl.ds(r0 + q * QT, QT)], stage)

        @pl.loop(0, QT // L)
        def _(i):
            v = plsc.load_gather(stage, [iota + i * L, zcol])
            cmp_[i] = v

        row_out = cid * NR + sid * (RPT // L) + q * (QT // L)
        pltpu.sync_copy(cmp_, deg_out.at[pl.ds(row_out, QT // L)])


def _dense1_body(x0_h, x1_h, deg_h, wvec_h, h1s_out, dis_out,
                 xb0, xb1, db0, db1, disb, hst, wbuf):
    cid, sid, wid = _ids()
    rw = wid * DR
    sl = pl.ds(rw, DR)
    pltpu.sync_copy(wvec_h, wbuf)
    pltpu.sync_copy(x0_h.at[sl], xb0)
    pltpu.sync_copy(x1_h.at[sl], xb1)
    pltpu.sync_copy(deg_h.at[sl], db0)
    pltpu.sync_copy(deg_h.at[pl.ds(NR + rw, DR)], db1)

    iota = lax.iota(_i32, L)
    wv0 = wbuf[0]
    _zero_fill(hst, DW, 8)

    @pl.loop(0, DR)
    def _(i):
        d = db0[i] + db1[i]
        y = _rsqrt16(d)
        disb[i] = y
        xv0 = xb0[i]
        xv1 = xb1[i]
        rows = iota + i * L
        for j in range(4):
            hj = (xv0 * wv0[j] + xv1 * wv0[4 + j]) * y
            plsc.store_scatter(hst, [rows, jnp.full((L,), j, _i32)], hj)

    pltpu.sync_copy(disb, dis_out.at[sl])
    pltpu.sync_copy(hst, h1s_out.at[pl.ds(wid * DW, DW)])


def _edgek_body(table_h, comb_h, acc_out,
                shared_acc, idx0, idx1, msg0, msg1, stage, sem0, sem1):
    cid, sid, wid = _ids()
    r0 = sid * RPT
    _zero_fill(stage, QT, 8)
    for q in range(4):
        pltpu.sync_copy(stage, shared_acc.at[pl.ds(r0 + q * QT, QT)])
    plsc.subcore_barrier()
    _edge_pass(comb_h, table_h, shared_acc, idx0, idx1, msg0, msg1,
               sem0, sem1, wid)
    plsc.subcore_barrier()
    _acc_writeback(shared_acc, stage, acc_out, r0, cid * NP + r0)


def _dense2_body(x0_h, x1_h, dis_h, acc1_h, wvec_h, h2s_out,
                 xb0, xb1, disb, a0b, a1b, h2st, wbuf):
    cid, sid, wid = _ids()
    rw = wid * DR
    sl = pl.ds(rw, DR)
    nb = wid * DW
    pltpu.sync_copy(wvec_h, wbuf)
    pltpu.sync_copy(x0_h.at[sl], xb0)
    pltpu.sync_copy(x1_h.at[sl], xb1)
    pltpu.sync_copy(dis_h.at[sl], disb)
    pltpu.sync_copy(acc1_h.at[pl.ds(nb, DW)], a0b)
    pltpu.sync_copy(acc1_h.at[pl.ds(NP + nb, DW)], a1b)

    iota = lax.iota(_i32, L)
    zero = jnp.zeros((L,), _f32)
    wv0 = wbuf[0]
    wv1 = wbuf[1]
    _zero_fill(h2st, DW, 8)

    @pl.loop(0, DR)
    def _(i):
        y = disb[i]
        xv0 = xb0[i]
        xv1 = xb1[i]
        rows = iota + i * L
        h2 = [zero, zero]
        for j in range(4):
            colj = jnp.full((L,), j, _i32)
            a = (plsc.load_gather(a0b, [rows, colj])
                 + plsc.load_gather(a1b, [rows, colj]))
            hj = (xv0 * wv0[j] + xv1 * wv0[4 + j]) * y
            o = jnp.maximum((a + hj) * y + wv1[j], 0.0)
            for k in range(2):
                h2[k] = h2[k] + o * wv0[8 + j * 2 + k]
        for k in range(2):
            plsc.store_scatter(h2st, [rows, jnp.full((L,), k, _i32)],
                               h2[k] * y)

    pltpu.sync_copy(h2st, h2s_out.at[pl.ds(nb, DW)])


def _final_body(acc2_h, h2s_h, dis_h, wvec_h, out_t,
                a0b, a1b, h2b, disb, st0, st1, wbuf):
    cid, sid, wid = _ids()
    rw = wid * DR
    nb = wid * DW
    pltpu.sync_copy(wvec_h, wbuf)
    pltpu.sync_copy(acc2_h.at[pl.ds(nb, DW)], a0b)
    pltpu.sync_copy(acc2_h.at[pl.ds(NP + nb, DW)], a1b)
    pltpu.sync_copy(h2s_h.at[pl.ds(nb, DW)], h2b)
    pltpu.sync_copy(dis_h.at[pl.ds(rw, DR)], disb)

    wv1 = wbuf[1]
    iota = lax.iota(_i32, L)
    stages = [st0, st1]

    @pl.loop(0, DR)
    def _(i):
        y = disb[i]
        rows = iota + i * L
        for k in range(2):
            colk = jnp.full((L,), k, _i32)
            v = (plsc.load_gather(a0b, [rows, colk])
                 + plsc.load_gather(a1b, [rows, colk])
                 + plsc.load_gather(h2b, [rows, colk])) * y + wv1[4 + k]
            stages[k][i] = v

    for k in range(2):
        pltpu.sync_copy(stages[k], out_t.at[k, pl.ds(rw, DR)])


_deg_kernel = pl.kernel(
    _deg_body,
    compiler_params=_params,
    out_type=jax.ShapeDtypeStruct((CORES * NR, L), _f32),
    mesh=_mesh,
    scratch_types=[
        pltpu.VMEM_SHARED((NP, 8), _f32),
        pltpu.VMEM((W, 8), _f32),
        pltpu.VMEM((KB, W), _i32),
        pltpu.VMEM((QT, 8), _f32),
        pltpu.VMEM((QT // L, L), _f32),
        pltpu.SemaphoreType.DMA,
    ],
)

_dense1_kernel = pl.kernel(
    _dense1_body,
    compiler_params=_params,
    out_type=(jax.ShapeDtypeStruct((NP, 8), _f32),
              jax.ShapeDtypeStruct((NR, L), _f32)),
    mesh=_mesh,
    scratch_types=[
        pltpu.VMEM((DR, L), _f32),
        pltpu.VMEM((DR, L), _f32),
        pltpu.VMEM((DR, L), _f32),
        pltpu.VMEM((DR, L), _f32),
        pltpu.VMEM((DR, L), _f32),
        pltpu.VMEM((DW, 8), _f32),
        pltpu.VMEM((2, L), _f32),
    ],
)

_edgek_kernel = pl.kernel(
    _edgek_body,
    compiler_params=_params,
    out_type=jax.ShapeDtypeStruct((CORES * NP, 8), _f32),
    mesh=_mesh,
    scratch_types=[
        pltpu.VMEM_SHARED((NP, 8), _f32),
        pltpu.VMEM((2 * KB, W), _i32),
        pltpu.VMEM((2 * KB, W), _i32),
        pltpu.VMEM((KB, W, 8), _f32),
        pltpu.VMEM((KB, W, 8), _f32),
        pltpu.VMEM((QT, 8), _f32),
        pltpu.SemaphoreType.DMA,
        pltpu.SemaphoreType.DMA,
    ],
)

_dense2_kernel = pl.kernel(
    _dense2_body,
    compiler_params=_params,
    out_type=jax.ShapeDtypeStruct((NP, 8), _f32),
    mesh=_mesh,
    scratch_types=[
        pltpu.VMEM((DR, L), _f32),
        pltpu.VMEM((DR, L), _f32),
        pltpu.VMEM((DR, L), _f32),
        pltpu.VMEM((DW, 8), _f32),
        pltpu.VMEM((DW, 8), _f32),
        pltpu.VMEM((DW, 8), _f32),
        pltpu.VMEM((2, L), _f32),
    ],
)

_final_kernel = pl.kernel(
    _final_body,
    compiler_params=_params,
    out_type=jax.ShapeDtypeStruct((2, NR, L), _f32),
    mesh=_mesh,
    scratch_types=[
        pltpu.VMEM((DW, 8), _f32),
        pltpu.VMEM((DW, 8), _f32),
        pltpu.VMEM((DW, 8), _f32),
        pltpu.VMEM((DR, L), _f32),
        pltpu.VMEM((DR, L), _f32),
        pltpu.VMEM((DR, L), _f32),
        pltpu.VMEM((2, L), _f32),
    ],
)


@jax.jit
def kernel(x, edge_index, W1, b1, W2, b2):
    ei = edge_index.astype(_i32)
    pad = jnp.full((EPAD - E,), N, _i32)
    srcw = jnp.concatenate([ei[0], pad]).reshape(NW * G, KB, W)
    dstw = jnp.concatenate([ei[1], pad]).reshape(NW * G, KB, W)
    comb = jnp.concatenate([srcw, dstw], axis=1).reshape(NW * G * 2 * KB, W)
    dst512 = dstw.reshape(NW * G * KB, W)

    xp = jnp.pad(x.astype(_f32), ((0, NP - N), (0, 0)))
    x0 = xp[:, 0].reshape(NR, L)
    x1 = xp[:, 1].reshape(NR, L)
    wvec = jnp.concatenate([
        W1.reshape(-1), W2.reshape(-1), b1.reshape(-1), b2.reshape(-1),
        jnp.zeros((10,), _f32)]).astype(_f32).reshape(2, L)

    deg2 = _deg_kernel(dst512)
    h1s, dis = _dense1_kernel(x0, x1, deg2, wvec)
    acc1 = _edgek_kernel(h1s, comb)
    h2s = _dense2_kernel(x0, x1, dis, acc1, wvec)
    acc2 = _edgek_kernel(h2s, comb)
    out_t = _final_kernel(acc2, h2s, dis, wvec)
    return out_t.reshape(2, NP).T[:N]


# speedup vs baseline: 1.1041x; 1.0400x over previous
import jax
import jax.numpy as jnp
from jax import lax
from jax.experimental import pallas as pl
from jax.experimental.pallas import tpu as pltpu
from jax.experimental.pallas import tpu_sc as plsc

N = 100000
E = 6400000

L = 16
SUB = 16
CORES = 2
NW = CORES * SUB
NP = 100352
NR = NP // L
RPT = NP // SUB
QT = RPT // 4
DW = NP // NW
DR = DW // L

W = 512
KB = 4
G = 98
EPW = W * KB * G
EPAD = EPW * NW

_mesh = plsc.VectorSubcoreMesh(core_axis_name="c", subcore_axis_name="s")
_params = pltpu.CompilerParams(needs_layout_passes=False,
                               use_tc_tiling_on_sc=False)
_f32 = jnp.float32
_i32 = jnp.int32


def _rsqrt16(d):
    i = plsc.bitcast(d, _i32)
    i = jnp.int32(0x5F3759DF) - (i >> 1)
    y = plsc.bitcast(i, _f32)
    for _ in range(3):
        y = y * (1.5 - 0.5 * d * y * y)
    return y


def _ids():
    cid = lax.axis_index("c")
    sid = lax.axis_index("s")
    return cid, sid, cid * SUB + sid


def _edge_pass(comb_h, table_h, shared_acc, idx0, idx1, msg0, msg1,
               sem0, sem1, wid):
    grp_base = wid * G

    def load_idx(g, idx):
        pltpu.sync_copy(comb_h.at[pl.ds((grp_base + g) * 2 * KB, 2 * KB)],
                        idx)

    def fire_gathers(idx, msg, sem):
        return [pltpu.async_copy(table_h.at[idx.at[j]], msg.at[j], sem)
                for j in range(KB)]

    def wait_gathers(idx, msg, sem):
        for j in range(KB):
            pltpu.make_async_copy(table_h.at[idx.at[j]], msg.at[j],
                                  sem).wait()

    def fire_scatters(idx, msg, sem):
        return [pltpu.async_copy(msg.at[j], shared_acc.at[idx.at[KB + j]],
                                 sem, add=True)
                for j in range(KB)]

    def wait(descs):
        for d in descs:
            d.wait()

    load_idx(0, idx0)
    fire_gathers(idx0, msg0, sem0)

    @pl.loop(0, G // 2)
    def _(t):
        ga = 2 * t
        load_idx(ga + 1, idx1)
        fire_gathers(idx1, msg1, sem1)
        wait_gathers(idx0, msg0, sem0)
        wait(fire_scatters(idx0, msg0, sem0))

        @pl.when(t < G // 2 - 1)
        def _():
            load_idx(ga + 2, idx0)
            fire_gathers(idx0, msg0, sem0)

        wait_gathers(idx1, msg1, sem1)
        wait(fire_scatters(idx1, msg1, sem1))


def _acc_writeback(shared_acc, stage, out_ref, r0, out_base):
    for q in range(4):
        pltpu.sync_copy(shared_acc.at[pl.ds(r0 + q * QT, QT)], stage)
        pltpu.sync_copy(stage, out_ref.at[pl.ds(out_base + q * QT, QT)])


def _zero_fill(stage, nrows, ncols):
    iota = lax.iota(_i32, L)
    zero = jnp.zeros((L,), _f32)
    nit = nrows * ncols // L

    @pl.loop(0, nit)
    def _(i):
        f = iota + i * L
        plsc.store_scatter(stage, [f // ncols, f % ncols], zero)


def _deg_body(dst_h, deg_out, shared_deg, ones8, idx_d, idx_d2, stage, cmp_, sem):
    cid, sid, wid = _ids()
    r0 = sid * RPT
    iota = lax.iota(_i32, L)

    _zero_fill(ones8, W, 8)
    one = jnp.full((L,), 1.0, _f32)
    zcol = jnp.zeros((L,), _i32)

    @pl.loop(0, W // L)
    def _(t):
        plsc.store_scatter(ones8, [iota + t * L, zcol], one)

    _zero_fill(stage, QT, 8)
    vinit = one * jnp.where(cid == 0, 1.0, 0.0).astype(_f32)

    @pl.loop(0, QT // L)
    def _(i):
        plsc.store_scatter(stage, [iota + i * L, zcol], vinit)

    for q in range(4):
        pltpu.sync_copy(stage, shared_deg.at[pl.ds(r0 + q * QT, QT)])
    plsc.subcore_barrier()

    row_base = wid * G * KB

    def fire_scat(idx):
        return [pltpu.async_copy(ones8, shared_deg.at[idx.at[j]], sem,
                                 add=True)
                for j in range(KB)]

    def wait(descs):
        for d in descs:
            d.wait()

    pltpu.sync_copy(dst_h.at[pl.ds(row_base, KB)], idx_d)

    @pl.loop(0, G // 2)
    def _(t):
        rb = row_base + 2 * t * KB
        s0 = fire_scat(idx_d)
        pltpu.sync_copy(dst_h.at[pl.ds(rb + KB, KB)], idx_d2)
        wait(s0)
        s1 = fire_scat(idx_d2)

        @pl.when(t < G // 2 - 1)
        def _():
            pltpu.sync_copy(dst_h.at[pl.ds(rb + 2 * KB, KB)], idx_d)

        wait(s1)

    plsc.subcore_barrier()

    for q in range(4):
        pltpu.sync_copy(shared_deg.at[pl.ds(r0 + q * QT, QT)], stage)

        @pl.loop(0, QT // L)
        def _(i):
            v = plsc.load_gather(stage, [iota + i * L, zcol])
            cmp_[i] = v

        row_out = cid * NR + sid * (RPT // L) + q * (QT // L)
        pltpu.sync_copy(cmp_, deg_out.at[pl.ds(row_out, QT // L)])


def _dense1_body(x0_h, x1_h, deg_h, wvec_h, h1s_out, dis_out,
                 xb0, xb1, db0, db1, disb, hst, wbuf):
    cid, sid, wid = _ids()
    rw = wid * DR
    sl = pl.ds(rw, DR)
    pltpu.sync_copy(wvec_h, wbuf)
    pltpu.sync_copy(x0_h.at[sl], xb0)
    pltpu.sync_copy(x1_h.at[sl], xb1)
    pltpu.sync_copy(deg_h.at[sl], db0)
    pltpu.sync_copy(deg_h.at[pl.ds(NR + rw, DR)], db1)

    iota = lax.iota(_i32, L)
    wv0 = wbuf[0]
    _zero_fill(hst, DW, 8)

    @pl.loop(0, DR)
    def _(i):
        d = db0[i] + db1[i]
        y = _rsqrt16(d)
        disb[i] = y
        xv0 = xb0[i]
        xv1 = xb1[i]
        rows = iota + i * L
        for j in range(4):
            hj = (xv0 * wv0[j] + xv1 * wv0[4 + j]) * y
            plsc.store_scatter(hst, [rows, jnp.full((L,), j, _i32)], hj)

    pltpu.sync_copy(disb, dis_out.at[sl])
    pltpu.sync_copy(hst, h1s_out.at[pl.ds(wid * DW, DW)])


def _edgek_body(table_h, comb_h, acc_out,
                shared_acc, idx0, idx1, msg0, msg1, stage, sem0, sem1):
    cid, sid, wid = _ids()
    r0 = sid * RPT
    _zero_fill(stage, QT, 8)
    for q in range(4):
        pltpu.sync_copy(stage, shared_acc.at[pl.ds(r0 + q * QT, QT)])
    plsc.subcore_barrier()
    _edge_pass(comb_h, table_h, shared_acc, idx0, idx1, msg0, msg1,
               sem0, sem1, wid)
    plsc.subcore_barrier()
    _acc_writeback(shared_acc, stage, acc_out, r0, cid * NP + r0)


def _dense2_body(x0_h, x1_h, dis_h, acc1_h, wvec_h, h2s_out,
                 xb0, xb1, disb, a0b, a1b, h2st, wbuf):
    cid, sid, wid = _ids()
    rw = wid * DR
    sl = pl.ds(rw, DR)
    nb = wid * DW
    pltpu.sync_copy(wvec_h, wbuf)
    pltpu.sync_copy(x0_h.at[sl], xb0)
    pltpu.sync_copy(x1_h.at[sl], xb1)
    pltpu.sync_copy(dis_h.at[sl], disb)
    pltpu.sync_copy(acc1_h.at[pl.ds(nb, DW)], a0b)
    pltpu.sync_copy(acc1_h.at[pl.ds(NP + nb, DW)], a1b)

    iota = lax.iota(_i32, L)
    zero = jnp.zeros((L,), _f32)
    wv0 = wbuf[0]
    wv1 = wbuf[1]
    _zero_fill(h2st, DW, 8)

    @pl.loop(0, DR)
    def _(i):
        y = disb[i]
        xv0 = xb0[i]
        xv1 = xb1[i]
        rows = iota + i * L
        h2 = [zero, zero]
        for j in range(4):
            colj = jnp.full((L,), j, _i32)
            a = (plsc.load_gather(a0b, [rows, colj])
                 + plsc.load_gather(a1b, [rows, colj]))
            hj = (xv0 * wv0[j] + xv1 * wv0[4 + j]) * y
            o = jnp.maximum((a + hj) * y + wv1[j], 0.0)
            for k in range(2):
                h2[k] = h2[k] + o * wv0[8 + j * 2 + k]
        for k in range(2):
            plsc.store_scatter(h2st, [rows, jnp.full((L,), k, _i32)],
                               h2[k] * y)

    pltpu.sync_copy(h2st, h2s_out.at[pl.ds(nb, DW)])


def _final_body(acc2_h, h2s_h, dis_h, wvec_h, out_t,
                a0b, a1b, h2b, disb, st0, st1, wbuf):
    cid, sid, wid = _ids()
    rw = wid * DR
    nb = wid * DW
    pltpu.sync_copy(wvec_h, wbuf)
    pltpu.sync_copy(acc2_h.at[pl.ds(nb, DW)], a0b)
    pltpu.sync_copy(acc2_h.at[pl.ds(NP + nb, DW)], a1b)
    pltpu.sync_copy(h2s_h.at[pl.ds(nb, DW)], h2b)
    pltpu.sync_copy(dis_h.at[pl.ds(rw, DR)], disb)

    wv1 = wbuf[1]
    iota = lax.iota(_i32, L)
    stages = [st0, st1]

    @pl.loop(0, DR)
    def _(i):
        y = disb[i]
        rows = iota + i * L
        for k in range(2):
            colk = jnp.full((L,), k, _i32)
            v = (plsc.load_gather(a0b, [rows, colk])
                 + plsc.load_gather(a1b, [rows, colk])
                 + plsc.load_gather(h2b, [rows, colk])) * y + wv1[4 + k]
            stages[k][i] = v

    for k in range(2):
        pltpu.sync_copy(stages[k], out_t.at[k, pl.ds(rw, DR)])


_deg_kernel = pl.kernel(
    _deg_body,
    compiler_params=_params,
    out_type=jax.ShapeDtypeStruct((CORES * NR, L), _f32),
    mesh=_mesh,
    scratch_types=[
        pltpu.VMEM_SHARED((NP, 8), _f32),
        pltpu.VMEM((W, 8), _f32),
        pltpu.VMEM((KB, W), _i32),
        pltpu.VMEM((KB, W), _i32),
        pltpu.VMEM((QT, 8), _f32),
        pltpu.VMEM((QT // L, L), _f32),
        pltpu.SemaphoreType.DMA,
    ],
)

_dense1_kernel = pl.kernel(
    _dense1_body,
    compiler_params=_params,
    out_type=(jax.ShapeDtypeStruct((NP, 8), _f32),
              jax.ShapeDtypeStruct((NR, L), _f32)),
    mesh=_mesh,
    scratch_types=[
        pltpu.VMEM((DR, L), _f32),
        pltpu.VMEM((DR, L), _f32),
        pltpu.VMEM((DR, L), _f32),
        pltpu.VMEM((DR, L), _f32),
        pltpu.VMEM((DR, L), _f32),
        pltpu.VMEM((DW, 8), _f32),
        pltpu.VMEM((2, L), _f32),
    ],
)

_edgek_kernel = pl.kernel(
    _edgek_body,
    compiler_params=_params,
    out_type=jax.ShapeDtypeStruct((CORES * NP, 8), _f32),
    mesh=_mesh,
    scratch_types=[
        pltpu.VMEM_SHARED((NP, 8), _f32),
        pltpu.VMEM((2 * KB, W), _i32),
        pltpu.VMEM((2 * KB, W), _i32),
        pltpu.VMEM((KB, W, 8), _f32),
        pltpu.VMEM((KB, W, 8), _f32),
        pltpu.VMEM((QT, 8), _f32),
        pltpu.SemaphoreType.DMA,
        pltpu.SemaphoreType.DMA,
    ],
)

_dense2_kernel = pl.kernel(
    _dense2_body,
    compiler_params=_params,
    out_type=jax.ShapeDtypeStruct((NP, 8), _f32),
    mesh=_mesh,
    scratch_types=[
        pltpu.VMEM((DR, L), _f32),
        pltpu.VMEM((DR, L), _f32),
        pltpu.VMEM((DR, L), _f32),
        pltpu.VMEM((DW, 8), _f32),
        pltpu.VMEM((DW, 8), _f32),
        pltpu.VMEM((DW, 8), _f32),
        pltpu.VMEM((2, L), _f32),
    ],
)

_final_kernel = pl.kernel(
    _final_body,
    compiler_params=_params,
    out_type=jax.ShapeDtypeStruct((2, NR, L), _f32),
    mesh=_mesh,
    scratch_types=[
        pltpu.VMEM((DW, 8), _f32),
        pltpu.VMEM((DW, 8), _f32),
        pltpu.VMEM((DW, 8), _f32),
        pltpu.VMEM((DR, L), _f32),
        pltpu.VMEM((DR, L), _f32),
        pltpu.VMEM((DR, L), _f32),
        pltpu.VMEM((2, L), _f32),
    ],
)


@jax.jit
def kernel(x, edge_index, W1, b1, W2, b2):
    ei = edge_index.astype(_i32)
    pad = jnp.full((EPAD - E,), N, _i32)
    srcw = jnp.concatenate([ei[0], pad]).reshape(NW * G, KB, W)
    dstw = jnp.concatenate([ei[1], pad]).reshape(NW * G, KB, W)
    comb = jnp.concatenate([srcw, dstw], axis=1).reshape(NW * G * 2 * KB, W)
    dst512 = dstw.reshape(NW * G * KB, W)

    xp = jnp.pad(x.astype(_f32), ((0, NP - N), (0, 0)))
    x0 = xp[:, 0].reshape(NR, L)
    x1 = xp[:, 1].reshape(NR, L)
    wvec = jnp.concatenate([
        W1.reshape(-1), W2.reshape(-1), b1.reshape(-1), b2.reshape(-1),
        jnp.zeros((10,), _f32)]).astype(_f32).reshape(2, L)

    deg2 = _deg_kernel(dst512)
    h1s, dis = _dense1_kernel(x0, x1, deg2, wvec)
    acc1 = _edgek_kernel(h1s, comb)
    h2s = _dense2_kernel(x0, x1, dis, acc1, wvec)
    acc2 = _edgek_kernel(h2s, comb)
    out_t = _final_kernel(acc2, h2s, dis, wvec)
    return out_t.reshape(2, NP).T[:N]
